# Initial kernel scaffold; baseline (speedup 1.0000x reference)
import functools

import jax
import jax.numpy as jnp
from jax import lax
from jax.experimental import pallas as pl
from jax.experimental.pallas import tpu as pltpu
from jax.experimental.pallas import tpu_sc as plsc

VOCAB = 100000
BATCH = 4096
HIST = 50

NC = 2
NS = 16
L = 16
NW = NC * NS
B = BATCH * HIST
B_PER_W = B // NW


def _body(idx_hbm, table_hbm, out_hbm, table_v, idx_v, out_v, sem):
    wid = lax.axis_index("s") * NC + lax.axis_index("c")
    base = wid * B_PER_W
    tcp = pltpu.async_copy(table_hbm, table_v, sem)
    pltpu.sync_copy(idx_hbm.at[pl.ds(base, B_PER_W)], idx_v)
    tcp.wait()

    def step(i, carry):
        off = i * L
        idx16 = idx_v[pl.ds(off, L)]
        out_v[pl.ds(off, L)] = plsc.load_gather(table_v, [idx16])
        return carry

    lax.fori_loop(0, B_PER_W // L, step, 0, unroll=4)
    pltpu.sync_copy(out_v, out_hbm.at[pl.ds(base, B_PER_W)])


@jax.jit
def _lookup(idx_flat, table):
    mesh = plsc.VectorSubcoreMesh(core_axis_name="c", subcore_axis_name="s")
    return pl.kernel(
        _body,
        out_type=jax.ShapeDtypeStruct((B,), jnp.float32),
        mesh=mesh,
        scratch_types=[
            pltpu.VMEM((VOCAB,), jnp.float32),
            pltpu.VMEM((B_PER_W,), jnp.int32),
            pltpu.VMEM((B_PER_W,), jnp.float32),
            pltpu.SemaphoreType.DMA,
        ],
    )(idx_flat, table)


def kernel(inputs, table):
    out = _lookup(inputs.reshape(B), table)
    return out.reshape(BATCH, HIST)


# speedup vs baseline: 35.7760x; 35.7760x over previous
import functools

import jax
import jax.numpy as jnp
from jax import lax
from jax.experimental import pallas as pl
from jax.experimental.pallas import tpu as pltpu
from jax.experimental.pallas import tpu_sc as plsc

VOCAB = 100000
BATCH = 4096
HIST = 50

NC = 2
NS = 16
L = 16
NW = NC * NS
B = BATCH * HIST
B_PER_W = B // NW


def _body(idx_hbm, table_hbm, out_hbm, table_v, idx_v, out_v, sem):
    wid = lax.axis_index("s") * NC + lax.axis_index("c")
    base = wid * B_PER_W
    tcp = pltpu.async_copy(table_hbm, table_v, sem)
    pltpu.sync_copy(idx_hbm.at[pl.ds(base, B_PER_W)], idx_v)
    tcp.wait()

    def step(i, carry):
        off = i * L
        idx16 = idx_v[pl.ds(off, L)]
        out_v[pl.ds(off, L)] = plsc.load_gather(table_v, [idx16])
        return carry

    lax.fori_loop(0, B_PER_W // L, step, 0, unroll=4)
    pltpu.sync_copy(out_v, out_hbm.at[pl.ds(base, B_PER_W)])


@jax.jit
def _lookup(idx_flat, table):
    mesh = plsc.VectorSubcoreMesh(core_axis_name="c", subcore_axis_name="s")
    return pl.kernel(
        _body,
        out_type=jax.ShapeDtypeStruct((B,), jnp.float32),
        mesh=mesh,
        compiler_params=pltpu.CompilerParams(needs_layout_passes=False),
        scratch_types=[
            pltpu.VMEM((VOCAB,), jnp.float32),
            pltpu.VMEM((B_PER_W,), jnp.int32),
            pltpu.VMEM((B_PER_W,), jnp.float32),
            pltpu.SemaphoreType.DMA,
        ],
    )(idx_flat, table)


def kernel(inputs, table):
    out = _lookup(inputs.reshape(B), table)
    return out.reshape(BATCH, HIST)


# speedup vs baseline: 51.6797x; 1.4445x over previous
import jax
import jax.numpy as jnp
from jax import lax
from jax.experimental import pallas as pl
from jax.experimental.pallas import tpu as pltpu
from jax.experimental.pallas import tpu_sc as plsc

VOCAB = 100000
BATCH = 4096
HIST = 50

NC = 2
NS = 16
L = 16
NW = NC * NS
B = BATCH * HIST
B_PER_W = B // NW


def _body(idx_hbm, table_hbm, out_hbm, table_sh, idx_v, out_v, sem, isem):
    cid = lax.axis_index("c")
    sid = lax.axis_index("s")
    wid = sid * NC + cid
    base = wid * B_PER_W

    icp = pltpu.async_copy(idx_hbm.at[pl.ds(base, B_PER_W)], idx_v, isem)

    @pl.when(sid == 0)
    def _():
        pltpu.sync_copy(table_hbm, table_sh)

    plsc.subcore_barrier()
    icp.wait()

    pltpu.async_copy(table_sh.at[idx_v], out_v, sem).wait()
    pltpu.sync_copy(out_v, out_hbm.at[pl.ds(base, B_PER_W)])


@jax.jit
def _lookup(idx_flat, table):
    mesh = plsc.VectorSubcoreMesh(core_axis_name="c", subcore_axis_name="s")
    return pl.kernel(
        _body,
        out_type=jax.ShapeDtypeStruct((B,), jnp.float32),
        mesh=mesh,
        compiler_params=pltpu.CompilerParams(needs_layout_passes=False),
        scratch_types=[
            pltpu.VMEM_SHARED((VOCAB,), jnp.float32),
            pltpu.VMEM((B_PER_W,), jnp.int32),
            pltpu.VMEM((B_PER_W,), jnp.float32),
            pltpu.SemaphoreType.DMA,
            pltpu.SemaphoreType.DMA,
        ],
    )(idx_flat, table)


def kernel(inputs, table):
    out = _lookup(inputs.reshape(B), table)
    return out.reshape(BATCH, HIST)


# speedup vs baseline: 54.8712x; 1.0618x over previous
import jax
import jax.numpy as jnp
from jax import lax
from jax.experimental import pallas as pl
from jax.experimental.pallas import tpu as pltpu
from jax.experimental.pallas import tpu_sc as plsc

VOCAB = 100000
BATCH = 4096
HIST = 50

NC = 2
NS = 16
NW = NC * NS
ROWS_PER_W = BATCH // NW


def _body(idx_hbm, table_hbm, out_hbm, table_sh, idx_v, out_v, sem, isem):
    cid = lax.axis_index("c")
    sid = lax.axis_index("s")
    wid = sid * NC + cid
    r0 = wid * ROWS_PER_W

    icp = pltpu.async_copy(idx_hbm.at[pl.ds(r0, ROWS_PER_W), :], idx_v, isem)

    @pl.when(sid == 0)
    def _():
        pltpu.sync_copy(table_hbm, table_sh)

    plsc.subcore_barrier()
    icp.wait()

    CHUNK = 8

    def step(c, carry):
        row = c * CHUNK
        cps = [
            pltpu.async_copy(
                table_sh.at[idx_v.at[row + j]], out_v.at[row + j], sem
            )
            for j in range(CHUNK)
        ]
        for cp in cps:
            cp.wait()
        return carry

    lax.fori_loop(0, ROWS_PER_W // CHUNK, step, 0)
    pltpu.sync_copy(out_v, out_hbm.at[pl.ds(r0, ROWS_PER_W), :])


@jax.jit
def kernel(inputs, table):
    mesh = plsc.VectorSubcoreMesh(core_axis_name="c", subcore_axis_name="s")
    return pl.kernel(
        _body,
        out_type=jax.ShapeDtypeStruct((BATCH, HIST), jnp.float32),
        mesh=mesh,
        compiler_params=pltpu.CompilerParams(needs_layout_passes=False),
        scratch_types=[
            pltpu.VMEM_SHARED((VOCAB,), jnp.float32),
            pltpu.VMEM((ROWS_PER_W, HIST), jnp.int32),
            pltpu.VMEM((ROWS_PER_W, HIST), jnp.float32),
            pltpu.SemaphoreType.DMA,
            pltpu.SemaphoreType.DMA,
        ],
    )(inputs, table)


# speedup vs baseline: 54.9150x; 1.0008x over previous
import jax
import jax.numpy as jnp
from jax import lax
from jax.experimental import pallas as pl
from jax.experimental.pallas import tpu as pltpu
from jax.experimental.pallas import tpu_sc as plsc

VOCAB = 100000
BATCH = 4096
HIST = 50

NC = 2
NS = 16
NW = NC * NS
ROWS_PER_W = BATCH // NW


def _body(idx_hbm, table_hbm, out_hbm, table_sh, idx_v, out_v, sem, isem):
    cid = lax.axis_index("c")
    sid = lax.axis_index("s")
    wid = sid * NC + cid
    r0 = wid * ROWS_PER_W

    icp = pltpu.async_copy(idx_hbm.at[pl.ds(r0, ROWS_PER_W), :], idx_v, isem)

    @pl.when(sid == 0)
    def _():
        pltpu.sync_copy(table_hbm, table_sh)

    plsc.subcore_barrier()
    icp.wait()

    CHUNK = 8

    def step(c, carry):
        row = c * CHUNK
        cps = [
            pltpu.async_copy(
                table_sh.at[idx_v.at[row + j]], out_v.at[row + j], sem
            )
            for j in range(CHUNK)
        ]
        for cp in cps:
            cp.wait()
        return carry

    lax.fori_loop(0, ROWS_PER_W // CHUNK, step, 0)
    pltpu.sync_copy(out_v, out_hbm.at[pl.ds(r0, ROWS_PER_W), :])


@jax.jit
def kernel(inputs, table):
    mesh = plsc.VectorSubcoreMesh(core_axis_name="c", subcore_axis_name="s")
    return pl.kernel(
        _body,
        out_type=jax.ShapeDtypeStruct((BATCH, HIST), jnp.float32),
        mesh=mesh,
        compiler_params=pltpu.CompilerParams(
            needs_layout_passes=False, use_tc_tiling_on_sc=True
        ),
        scratch_types=[
            pltpu.VMEM_SHARED((VOCAB,), jnp.float32),
            pltpu.VMEM((ROWS_PER_W, HIST), jnp.int32),
            pltpu.VMEM((ROWS_PER_W, HIST), jnp.float32),
            pltpu.SemaphoreType.DMA,
            pltpu.SemaphoreType.DMA,
        ],
    )(inputs, table)


# speedup vs baseline: 66.1812x; 1.2052x over previous
import jax
import jax.numpy as jnp
from jax import lax
from jax.experimental import pallas as pl
from jax.experimental.pallas import tpu as pltpu
from jax.experimental.pallas import tpu_sc as plsc

VOCAB = 100000
BATCH = 4096
HIST = 50

NC = 2
NS = 16
NW = NC * NS
COLS_PER_W = BATCH // NW
CHUNK = 10


def _body(idx_hbm, table_hbm, out_hbm, table_sh, idx_v, out_v, sem, isem):
    cid = lax.axis_index("c")
    sid = lax.axis_index("s")
    wid = sid * NC + cid
    c0 = wid * COLS_PER_W

    icp = pltpu.async_copy(idx_hbm.at[:, pl.ds(c0, COLS_PER_W)], idx_v, isem)

    @pl.when(sid == 0)
    def _():
        pltpu.sync_copy(table_hbm, table_sh)

    plsc.subcore_barrier()
    icp.wait()

    def step(c, carry):
        row = c * CHUNK
        cps = [
            pltpu.async_copy(
                table_sh.at[idx_v.at[row + j]], out_v.at[row + j], sem
            )
            for j in range(CHUNK)
        ]
        for cp in cps:
            cp.wait()
        return carry

    lax.fori_loop(0, HIST // CHUNK, step, 0)
    pltpu.sync_copy(out_v, out_hbm.at[:, pl.ds(c0, COLS_PER_W)])


@jax.jit
def kernel(inputs, table):
    mesh = plsc.VectorSubcoreMesh(core_axis_name="c", subcore_axis_name="s")
    out_t = pl.kernel(
        _body,
        out_type=jax.ShapeDtypeStruct((HIST, BATCH), jnp.float32),
        mesh=mesh,
        compiler_params=pltpu.CompilerParams(needs_layout_passes=False),
        scratch_types=[
            pltpu.VMEM_SHARED((VOCAB,), jnp.float32),
            pltpu.VMEM((HIST, COLS_PER_W), jnp.int32),
            pltpu.VMEM((HIST, COLS_PER_W), jnp.float32),
            pltpu.SemaphoreType.DMA,
            pltpu.SemaphoreType.DMA,
        ],
    )(inputs.T, table)
    return out_t.T
